# baseline (speedup 1.0000x reference)
import functools

import jax
import jax.numpy as jnp
import numpy as np
from jax import lax
from jax.experimental import pallas as pl
from jax.experimental.pallas import tpu as pltpu
from jax.experimental.pallas import tpu_sc as plsc

N = 10000
M = 10000
NNZ = 320000
HID = 64
EPS = 1e-5

NC = 2
NS = 16
NW = NC * NS
PER_W = NNZ // NW
CHUNK = 128
PER_W_PAD = 10240
PAD = PER_W_PAD - PER_W
NCHUNK = PER_W_PAD // CHUNK
NBUF = 4
MP = 10240
ROWS_PER_TILE = MP // NS
SUB = 128
NSUB = ROWS_PER_TILE // SUB

f32 = jnp.float32

_PAD_IDX = (N + (np.arange(NW, dtype=np.int32)[:, None]
                 + np.arange(PAD, dtype=np.int32)[None, :]) % (MP - N))


def _mesh():
    return plsc.VectorSubcoreMesh(
        core_axis_name="c", subcore_axis_name="s", num_cores=NC, num_subcores=NS
    )


def _sc_compiler_params():
    return pltpu.CompilerParams(use_tc_tiling_on_sc=False)


def _ring_scratch():
    return ([pltpu.VMEM((NCHUNK, CHUNK), jnp.int32)] * 2
            + [pltpu.VMEM((CHUNK, HID), f32)] * NBUF
            + [pltpu.VMEM_SHARED((MP, HID), f32)]
            + [pltpu.SemaphoreType.DMA] * (2 * NBUF))


def _run_ring(table_hbm, gix_v, six_v, bufs, acc, gsems, ssems):
    def start_gather(ci, b):
        pltpu.async_copy(table_hbm.at[gix_v.at[ci]], bufs[b], gsems[b])

    def wait_gather(b):
        pltpu.make_async_copy(table_hbm.at[gix_v.at[0]], bufs[b],
                              gsems[b]).wait()

    def start_scatter(ci, b):
        pltpu.async_copy(bufs[b], acc.at[six_v.at[ci]], ssems[b], add=True)

    def wait_scatter(b):
        pltpu.make_async_copy(bufs[b], acc.at[six_v.at[0]], ssems[b]).wait()

    for b in range(NBUF):
        start_gather(b, b)

    @pl.loop(0, NCHUNK, step=NBUF)
    def _(ci):
        for b in range(NBUF):
            wait_gather(b)
            start_scatter(ci + b, b)
        for b in range(NBUF):
            wait_scatter(b)
            start_gather(jnp.minimum(ci + NBUF + b, NCHUNK - 1), b)

    for b in range(NBUF):
        wait_gather(b)


@functools.cache
def _stage1_call():
    @functools.partial(
        pl.kernel,
        out_type=jax.ShapeDtypeStruct((NC, MP, HID), f32),
        mesh=_mesh(),
        compiler_params=_sc_compiler_params(),
        scratch_types=_ring_scratch(),
    )
    def stage1(table_hbm, gidx_hbm, sidx_hbm, zeros_hbm, out_hbm,
               gix_v, six_v, *rest):
        bufs = rest[:NBUF]
        acc = rest[NBUF]
        gsems = rest[NBUF + 1:NBUF + 1 + NBUF]
        ssems = rest[NBUF + 1 + NBUF:]
        cid = lax.axis_index("c")
        sid = lax.axis_index("s")
        wid = cid * NS + sid
        base = sid * ROWS_PER_TILE
        pltpu.sync_copy(zeros_hbm.at[pl.ds(base, ROWS_PER_TILE)],
                        acc.at[pl.ds(base, ROWS_PER_TILE)])
        pltpu.sync_copy(gidx_hbm.at[wid], gix_v)
        pltpu.sync_copy(sidx_hbm.at[wid], six_v)
        plsc.subcore_barrier()
        _run_ring(table_hbm, gix_v, six_v, bufs, acc, gsems, ssems)
        plsc.subcore_barrier()
        pltpu.sync_copy(acc.at[pl.ds(base, ROWS_PER_TILE)],
                        out_hbm.at[cid, pl.ds(base, ROWS_PER_TILE)])

    return stage1


@functools.cache
def _stage2_call():
    @functools.partial(
        pl.kernel,
        out_type=[jax.ShapeDtypeStruct((NC, MP, HID), f32),
                  jax.ShapeDtypeStruct((MP, HID), f32),
                  jax.ShapeDtypeStruct((MP, HID), f32)],
        mesh=_mesh(),
        compiler_params=_sc_compiler_params(),
        scratch_types=_ring_scratch() + [
            pltpu.VMEM((SUB, HID), f32),
            pltpu.VMEM((SUB, HID), f32),
            pltpu.VMEM((SUB, HID), f32),
            pltpu.VMEM((SUB, 16), f32),
            pltpu.VMEM((SUB, 16), f32),
        ],
    )
    def stage2(pe_hbm, ecnt_hbm, gidx_hbm, sidx_hbm, zeros_hbm,
               out_hbm, tbl0_hbm, tbl1_hbm,
               gix_v, six_v, *rest):
        bufs = rest[:NBUF]
        acc = rest[NBUF]
        gsems = rest[NBUF + 1:NBUF + 1 + NBUF]
        ssems = rest[NBUF + 1 + NBUF:NBUF + 1 + 2 * NBUF]
        pa, pb, tr, ca, cb = rest[NBUF + 1 + 2 * NBUF:]
        cid = lax.axis_index("c")
        sid = lax.axis_index("s")
        wid = cid * NS + sid
        base = sid * ROWS_PER_TILE
        pltpu.sync_copy(zeros_hbm.at[pl.ds(base, ROWS_PER_TILE)],
                        acc.at[pl.ds(base, ROWS_PER_TILE)])
        pltpu.sync_copy(gidx_hbm.at[wid], gix_v)
        pltpu.sync_copy(sidx_hbm.at[wid], six_v)

        for t in range(NSUB):
            r0 = base + t * SUB
            pltpu.sync_copy(pe_hbm.at[0, pl.ds(r0, SUB)], pa)
            pltpu.sync_copy(pe_hbm.at[1, pl.ds(r0, SUB)], pb)
            pltpu.sync_copy(ecnt_hbm.at[0, pl.ds(r0, SUB)], ca)
            pltpu.sync_copy(ecnt_hbm.at[1, pl.ds(r0, SUB)], cb)

            @pl.loop(0, SUB, unroll=8)
            def _(r):
                deg = ca[r] + cb[r]
                binv = jnp.where(deg > 0, 1.0 / deg, 0.0)
                for k in range(HID // 16):
                    s = pl.ds(k * 16, 16)
                    tr[r, s] = (pa[r, s] + pb[r, s]) * binv

            @pl.when(cid == 0)
            def _():
                pltpu.sync_copy(tr, tbl0_hbm.at[pl.ds(r0, SUB)])

            @pl.when(cid == 1)
            def _():
                pltpu.sync_copy(tr, tbl1_hbm.at[pl.ds(r0, SUB)])

        plsc.subcore_barrier()

        @pl.when(cid == 0)
        def _():
            _run_ring(tbl0_hbm, gix_v, six_v, bufs, acc, gsems, ssems)

        @pl.when(cid == 1)
        def _():
            _run_ring(tbl1_hbm, gix_v, six_v, bufs, acc, gsems, ssems)

        plsc.subcore_barrier()
        pltpu.sync_copy(acc.at[pl.ds(base, ROWS_PER_TILE)],
                        out_hbm.at[cid, pl.ds(base, ROWS_PER_TILE)])

    return stage2


@functools.cache
def _hist_call():
    @functools.partial(
        pl.kernel,
        out_type=[jax.ShapeDtypeStruct((NC, MP, 16), f32),
                  jax.ShapeDtypeStruct((NC, MP, 16), f32)],
        mesh=_mesh(),
        compiler_params=_sc_compiler_params(),
        scratch_types=[
            pltpu.VMEM((NCHUNK, CHUNK), jnp.int32),
            pltpu.VMEM((NCHUNK, CHUNK), jnp.int32),
            pltpu.VMEM((CHUNK, 16), f32),
            pltpu.VMEM_SHARED((MP, 16), f32),
            pltpu.VMEM_SHARED((MP, 16), f32),
            pltpu.SemaphoreType.DMA,
            pltpu.SemaphoreType.DMA,
        ],
    )
    def hist(src_hbm, he_hbm, zeros16_hbm, ones_hbm, dout_hbm, eout_hbm,
             src_v, he_v, ones_v, dacc, eacc, sem_d, sem_e):
        cid = lax.axis_index("c")
        sid = lax.axis_index("s")
        wid = cid * NS + sid
        base = sid * ROWS_PER_TILE
        pltpu.sync_copy(zeros16_hbm.at[pl.ds(base, ROWS_PER_TILE)],
                        dacc.at[pl.ds(base, ROWS_PER_TILE)])
        pltpu.sync_copy(zeros16_hbm.at[pl.ds(base, ROWS_PER_TILE)],
                        eacc.at[pl.ds(base, ROWS_PER_TILE)])
        pltpu.sync_copy(ones_hbm, ones_v)
        pltpu.sync_copy(src_hbm.at[wid], src_v)
        pltpu.sync_copy(he_hbm.at[wid], he_v)
        plsc.subcore_barrier()

        K = 5

        @pl.loop(0, NCHUNK, step=K)
        def _(ci):
            for j in range(K):
                pltpu.async_copy(ones_v, dacc.at[src_v.at[ci + j]], sem_d,
                                 add=True)
                pltpu.async_copy(ones_v, eacc.at[he_v.at[ci + j]], sem_e,
                                 add=True)
            for j in range(K):
                pltpu.make_async_copy(ones_v, dacc.at[src_v.at[ci]],
                                      sem_d).wait()
                pltpu.make_async_copy(ones_v, eacc.at[he_v.at[ci]],
                                      sem_e).wait()

        plsc.subcore_barrier()
        pltpu.sync_copy(dacc.at[pl.ds(base, ROWS_PER_TILE)],
                        dout_hbm.at[cid, pl.ds(base, ROWS_PER_TILE)])
        pltpu.sync_copy(eacc.at[pl.ds(base, ROWS_PER_TILE)],
                        eout_hbm.at[cid, pl.ds(base, ROWS_PER_TILE)])

    return hist


BN_ROWS = 1024


def _tc_matmul0(x, w):
    def body(x_ref, w_ref, o_ref):
        o_ref[...] = jnp.dot(x_ref[...], w_ref[...],
                             preferred_element_type=f32)

    d_in = x.shape[1]
    return pl.pallas_call(
        body,
        grid=(MP // BN_ROWS,),
        in_specs=[
            pl.BlockSpec((BN_ROWS, d_in), lambda i: (i, 0)),
            pl.BlockSpec((d_in, HID), lambda i: (0, 0)),
        ],
        out_specs=pl.BlockSpec((BN_ROWS, HID), lambda i: (i, 0)),
        out_shape=jax.ShapeDtypeStruct((MP, HID), f32),
    )(x, w)


def _tc_epilogue_matmul(pn, dcnt, b, g, be, rm, rv, w):
    def body(pn_ref, cnt_ref, b_ref, g_ref, be_ref, rm_ref, rv_ref, w_ref,
             o_ref):
        s = pn_ref[0] + pn_ref[1]
        deg = cnt_ref[0, :, 0:1] + cnt_ref[1, :, 0:1]
        dinv = jnp.where(deg > 0, 1.0 / deg, 0.0)
        scale = g_ref[...] * lax.rsqrt(rv_ref[...] + EPS)
        shift = (b_ref[...] - rm_ref[...]) * scale + be_ref[...]
        h = jnp.maximum(s * dinv * scale + shift, 0.0)
        o_ref[...] = jnp.dot(h, w_ref[...], preferred_element_type=f32)

    vec = lambda: pl.BlockSpec((1, HID), lambda i: (0, 0))
    return pl.pallas_call(
        body,
        grid=(MP // BN_ROWS,),
        in_specs=[
            pl.BlockSpec((NC, BN_ROWS, HID), lambda i: (0, i, 0)),
            pl.BlockSpec((NC, BN_ROWS, 16), lambda i: (0, i, 0)),
            vec(), vec(), vec(), vec(), vec(),
            pl.BlockSpec((HID, HID), lambda i: (0, 0)),
        ],
        out_specs=pl.BlockSpec((BN_ROWS, HID), lambda i: (i, 0)),
        out_shape=jax.ShapeDtypeStruct((MP, HID), f32),
    )(pn, dcnt, b.reshape(1, HID), g.reshape(1, HID), be.reshape(1, HID),
      rm.reshape(1, HID), rv.reshape(1, HID), w)


def _tc_epilogue_head(pn, dcnt, b, g, be, rm, rv, wc1, bc1, wc2, bc2):
    h1 = wc1.shape[1]
    ncls = wc2.shape[1]
    rows = 1000

    def body(pn_ref, cnt_ref, b_ref, g_ref, be_ref, rm_ref, rv_ref,
             wc1_ref, bc1_ref, wc2_ref, bc2_ref, o_ref):
        s = pn_ref[0] + pn_ref[1]
        deg = cnt_ref[0, :, 0:1] + cnt_ref[1, :, 0:1]
        dinv = jnp.where(deg > 0, 1.0 / deg, 0.0)
        scale = g_ref[...] * lax.rsqrt(rv_ref[...] + EPS)
        shift = (b_ref[...] - rm_ref[...]) * scale + be_ref[...]
        h = jnp.maximum(s * dinv * scale + shift, 0.0)
        t = jnp.maximum(
            jnp.dot(h, wc1_ref[...], preferred_element_type=f32)
            + bc1_ref[...], 0.0)
        o_ref[...] = (jnp.dot(t, wc2_ref[...], preferred_element_type=f32)
                      + bc2_ref[...])

    vec = lambda: pl.BlockSpec((1, HID), lambda i: (0, 0))
    return pl.pallas_call(
        body,
        grid=(N // rows,),
        in_specs=[
            pl.BlockSpec((NC, rows, HID), lambda i: (0, i, 0)),
            pl.BlockSpec((NC, rows, 16), lambda i: (0, i, 0)),
            vec(), vec(), vec(), vec(), vec(),
            pl.BlockSpec((HID, h1), lambda i: (0, 0)),
            pl.BlockSpec((1, h1), lambda i: (0, 0)),
            pl.BlockSpec((h1, ncls), lambda i: (0, 0)),
            pl.BlockSpec((1, ncls), lambda i: (0, 0)),
        ],
        out_specs=pl.BlockSpec((rows, ncls), lambda i: (i, 0)),
        out_shape=jax.ShapeDtypeStruct((N, ncls), f32),
    )(pn, dcnt, b.reshape(1, HID), g.reshape(1, HID), be.reshape(1, HID),
      rm.reshape(1, HID), rv.reshape(1, HID), wc1, bc1.reshape(1, h1),
      wc2, bc2.reshape(1, ncls))


def kernel(x, hyperedge_index, W0, b0, g0, be0, rm0, rv0,
           W1, b1, g1, be1, rm1, rv1, W2, b2, g2, be2, rm2, rv2,
           Wc1, bc1, Wc2, bc2):
    pad = jnp.asarray(_PAD_IDX)

    def prep(idx):
        return jnp.concatenate([idx.reshape(NW, PER_W), pad], axis=1).reshape(
            NW, NCHUNK, CHUNK)

    p_src = prep(hyperedge_index[0])
    p_he = prep(hyperedge_index[1])
    zeros64 = jnp.zeros((MP, HID), f32)
    zeros16 = jnp.zeros((MP, 16), f32)
    ones16 = jnp.ones((CHUNK, 16), f32)

    dcnt, ecnt = _hist_call()(p_src, p_he, zeros16, ones16)
    stage1 = _stage1_call()
    stage2 = _stage2_call()

    params = [(b0, g0, be0, rm0, rv0),
              (b1, g1, be1, rm1, rv1),
              (b2, g2, be2, rm2, rv2)]
    next_w = [W1, W2]
    xw = _tc_matmul0(x, W0)
    for li, (b, g, be, rm, rv) in enumerate(params):
        pe = stage1(xw, p_src, p_he, zeros64)
        pn, _, _ = stage2(pe, ecnt, p_he, p_src, zeros64)
        if li < 2:
            xw = _tc_epilogue_matmul(pn, dcnt, b, g, be, rm, rv, next_w[li])
        else:
            return _tc_epilogue_head(pn, dcnt, b, g, be, rm, rv,
                                     Wc1, bc1, Wc2, bc2)


# speedup vs baseline: 1.0488x; 1.0488x over previous
import functools

import jax
import jax.numpy as jnp
import numpy as np
from jax import lax
from jax.experimental import pallas as pl
from jax.experimental.pallas import tpu as pltpu
from jax.experimental.pallas import tpu_sc as plsc

N = 10000
M = 10000
NNZ = 320000
HID = 64
EPS = 1e-5

NC = 2
NS = 16
NW = NC * NS
PER_W = NNZ // NW
CHUNK = 128
PER_W_PAD = 10240
PAD = PER_W_PAD - PER_W
NCHUNK = PER_W_PAD // CHUNK
NBUF = 4
MP = 10240
ROWS_PER_TILE = MP // NS
SUB = 128
NSUB = ROWS_PER_TILE // SUB

f32 = jnp.float32

_PAD_IDX = (N + (np.arange(NW, dtype=np.int32)[:, None]
                 + np.arange(PAD, dtype=np.int32)[None, :]) % (MP - N))


def _mesh():
    return plsc.VectorSubcoreMesh(
        core_axis_name="c", subcore_axis_name="s", num_cores=NC, num_subcores=NS
    )


def _sc_compiler_params():
    return pltpu.CompilerParams(use_tc_tiling_on_sc=False)


def _ring_scratch():
    return ([pltpu.VMEM((NCHUNK, CHUNK), jnp.int32)] * 2
            + [pltpu.VMEM((CHUNK, HID), f32)] * NBUF
            + [pltpu.VMEM_SHARED((MP, HID), f32)]
            + [pltpu.SemaphoreType.DMA] * (2 * NBUF))


def _run_ring(table_hbm, gix_v, six_v, bufs, acc, gsems, ssems):
    def start_gather(ci, b):
        pltpu.async_copy(table_hbm.at[gix_v.at[ci]], bufs[b], gsems[b])

    def wait_gather(b):
        pltpu.make_async_copy(table_hbm.at[gix_v.at[0]], bufs[b],
                              gsems[b]).wait()

    def start_scatter(ci, b):
        pltpu.async_copy(bufs[b], acc.at[six_v.at[ci]], ssems[b], add=True)

    def wait_scatter(b):
        pltpu.make_async_copy(bufs[b], acc.at[six_v.at[0]], ssems[b]).wait()

    for b in range(NBUF):
        start_gather(b, b)

    @pl.loop(0, NCHUNK, step=NBUF)
    def _(ci):
        for b in range(NBUF):
            wait_gather(b)
            start_scatter(ci + b, b)
        for b in range(NBUF):
            wait_scatter(b)
            start_gather(jnp.minimum(ci + NBUF + b, NCHUNK - 1), b)

    for b in range(NBUF):
        wait_gather(b)


@functools.cache
def _stage1_call():
    @functools.partial(
        pl.kernel,
        out_type=jax.ShapeDtypeStruct((NC, MP, HID), f32),
        mesh=_mesh(),
        compiler_params=_sc_compiler_params(),
        scratch_types=_ring_scratch(),
    )
    def stage1(table_hbm, gidx_hbm, sidx_hbm, zeros_hbm, out_hbm,
               gix_v, six_v, *rest):
        bufs = rest[:NBUF]
        acc = rest[NBUF]
        gsems = rest[NBUF + 1:NBUF + 1 + NBUF]
        ssems = rest[NBUF + 1 + NBUF:]
        cid = lax.axis_index("c")
        sid = lax.axis_index("s")
        wid = cid * NS + sid
        base = sid * ROWS_PER_TILE
        pltpu.sync_copy(zeros_hbm.at[pl.ds(base, ROWS_PER_TILE)],
                        acc.at[pl.ds(base, ROWS_PER_TILE)])
        pltpu.sync_copy(gidx_hbm.at[wid], gix_v)
        pltpu.sync_copy(sidx_hbm.at[wid], six_v)
        plsc.subcore_barrier()
        _run_ring(table_hbm, gix_v, six_v, bufs, acc, gsems, ssems)
        plsc.subcore_barrier()
        pltpu.sync_copy(acc.at[pl.ds(base, ROWS_PER_TILE)],
                        out_hbm.at[cid, pl.ds(base, ROWS_PER_TILE)])

    return stage1


@functools.cache
def _hist_call():
    @functools.partial(
        pl.kernel,
        out_type=[jax.ShapeDtypeStruct((NC, MP, 16), f32),
                  jax.ShapeDtypeStruct((NC, MP, 16), f32)],
        mesh=_mesh(),
        compiler_params=_sc_compiler_params(),
        scratch_types=[
            pltpu.VMEM((NCHUNK, CHUNK), jnp.int32),
            pltpu.VMEM((NCHUNK, CHUNK), jnp.int32),
            pltpu.VMEM((CHUNK, 16), f32),
            pltpu.VMEM_SHARED((MP, 16), f32),
            pltpu.VMEM_SHARED((MP, 16), f32),
            pltpu.SemaphoreType.DMA,
            pltpu.SemaphoreType.DMA,
        ],
    )
    def hist(src_hbm, he_hbm, zeros16_hbm, ones_hbm, dout_hbm, eout_hbm,
             src_v, he_v, ones_v, dacc, eacc, sem_d, sem_e):
        cid = lax.axis_index("c")
        sid = lax.axis_index("s")
        wid = cid * NS + sid
        base = sid * ROWS_PER_TILE
        pltpu.sync_copy(zeros16_hbm.at[pl.ds(base, ROWS_PER_TILE)],
                        dacc.at[pl.ds(base, ROWS_PER_TILE)])
        pltpu.sync_copy(zeros16_hbm.at[pl.ds(base, ROWS_PER_TILE)],
                        eacc.at[pl.ds(base, ROWS_PER_TILE)])
        pltpu.sync_copy(ones_hbm, ones_v)
        pltpu.sync_copy(src_hbm.at[wid], src_v)
        pltpu.sync_copy(he_hbm.at[wid], he_v)
        plsc.subcore_barrier()

        K = 5

        @pl.loop(0, NCHUNK, step=K)
        def _(ci):
            for j in range(K):
                pltpu.async_copy(ones_v, dacc.at[src_v.at[ci + j]], sem_d,
                                 add=True)
                pltpu.async_copy(ones_v, eacc.at[he_v.at[ci + j]], sem_e,
                                 add=True)
            for j in range(K):
                pltpu.make_async_copy(ones_v, dacc.at[src_v.at[ci]],
                                      sem_d).wait()
                pltpu.make_async_copy(ones_v, eacc.at[he_v.at[ci]],
                                      sem_e).wait()

        plsc.subcore_barrier()
        pltpu.sync_copy(dacc.at[pl.ds(base, ROWS_PER_TILE)],
                        dout_hbm.at[cid, pl.ds(base, ROWS_PER_TILE)])
        pltpu.sync_copy(eacc.at[pl.ds(base, ROWS_PER_TILE)],
                        eout_hbm.at[cid, pl.ds(base, ROWS_PER_TILE)])

    return hist


BN_ROWS = 1024


def _tc_combine_e(pe, ecnt):
    def body(pe_ref, cnt_ref, o_ref):
        s = pe_ref[0] + pe_ref[1]
        edeg = cnt_ref[0, :, 0:1] + cnt_ref[1, :, 0:1]
        binv = jnp.where(edeg > 0, 1.0 / edeg, 0.0)
        o_ref[...] = s * binv

    return pl.pallas_call(
        body,
        grid=(MP // BN_ROWS,),
        in_specs=[
            pl.BlockSpec((NC, BN_ROWS, HID), lambda i: (0, i, 0)),
            pl.BlockSpec((NC, BN_ROWS, 16), lambda i: (0, i, 0)),
        ],
        out_specs=pl.BlockSpec((BN_ROWS, HID), lambda i: (i, 0)),
        out_shape=jax.ShapeDtypeStruct((MP, HID), f32),
    )(pe, ecnt)


def _tc_matmul0(x, w):
    def body(x_ref, w_ref, o_ref):
        o_ref[...] = jnp.dot(x_ref[...], w_ref[...],
                             preferred_element_type=f32)

    d_in = x.shape[1]
    return pl.pallas_call(
        body,
        grid=(MP // BN_ROWS,),
        in_specs=[
            pl.BlockSpec((BN_ROWS, d_in), lambda i: (i, 0)),
            pl.BlockSpec((d_in, HID), lambda i: (0, 0)),
        ],
        out_specs=pl.BlockSpec((BN_ROWS, HID), lambda i: (i, 0)),
        out_shape=jax.ShapeDtypeStruct((MP, HID), f32),
    )(x, w)


def _tc_epilogue_matmul(pn, dcnt, b, g, be, rm, rv, w):
    def body(pn_ref, cnt_ref, b_ref, g_ref, be_ref, rm_ref, rv_ref, w_ref,
             o_ref):
        s = pn_ref[0] + pn_ref[1]
        deg = cnt_ref[0, :, 0:1] + cnt_ref[1, :, 0:1]
        dinv = jnp.where(deg > 0, 1.0 / deg, 0.0)
        scale = g_ref[...] * lax.rsqrt(rv_ref[...] + EPS)
        shift = (b_ref[...] - rm_ref[...]) * scale + be_ref[...]
        h = jnp.maximum(s * dinv * scale + shift, 0.0)
        o_ref[...] = jnp.dot(h, w_ref[...], preferred_element_type=f32)

    vec = lambda: pl.BlockSpec((1, HID), lambda i: (0, 0))
    return pl.pallas_call(
        body,
        grid=(MP // BN_ROWS,),
        in_specs=[
            pl.BlockSpec((NC, BN_ROWS, HID), lambda i: (0, i, 0)),
            pl.BlockSpec((NC, BN_ROWS, 16), lambda i: (0, i, 0)),
            vec(), vec(), vec(), vec(), vec(),
            pl.BlockSpec((HID, HID), lambda i: (0, 0)),
        ],
        out_specs=pl.BlockSpec((BN_ROWS, HID), lambda i: (i, 0)),
        out_shape=jax.ShapeDtypeStruct((MP, HID), f32),
    )(pn, dcnt, b.reshape(1, HID), g.reshape(1, HID), be.reshape(1, HID),
      rm.reshape(1, HID), rv.reshape(1, HID), w)


def _tc_epilogue_head(pn, dcnt, b, g, be, rm, rv, wc1, bc1, wc2, bc2):
    h1 = wc1.shape[1]
    ncls = wc2.shape[1]
    rows = 1000

    def body(pn_ref, cnt_ref, b_ref, g_ref, be_ref, rm_ref, rv_ref,
             wc1_ref, bc1_ref, wc2_ref, bc2_ref, o_ref):
        s = pn_ref[0] + pn_ref[1]
        deg = cnt_ref[0, :, 0:1] + cnt_ref[1, :, 0:1]
        dinv = jnp.where(deg > 0, 1.0 / deg, 0.0)
        scale = g_ref[...] * lax.rsqrt(rv_ref[...] + EPS)
        shift = (b_ref[...] - rm_ref[...]) * scale + be_ref[...]
        h = jnp.maximum(s * dinv * scale + shift, 0.0)
        t = jnp.maximum(
            jnp.dot(h, wc1_ref[...], preferred_element_type=f32)
            + bc1_ref[...], 0.0)
        o_ref[...] = (jnp.dot(t, wc2_ref[...], preferred_element_type=f32)
                      + bc2_ref[...])

    vec = lambda: pl.BlockSpec((1, HID), lambda i: (0, 0))
    return pl.pallas_call(
        body,
        grid=(N // rows,),
        in_specs=[
            pl.BlockSpec((NC, rows, HID), lambda i: (0, i, 0)),
            pl.BlockSpec((NC, rows, 16), lambda i: (0, i, 0)),
            vec(), vec(), vec(), vec(), vec(),
            pl.BlockSpec((HID, h1), lambda i: (0, 0)),
            pl.BlockSpec((1, h1), lambda i: (0, 0)),
            pl.BlockSpec((h1, ncls), lambda i: (0, 0)),
            pl.BlockSpec((1, ncls), lambda i: (0, 0)),
        ],
        out_specs=pl.BlockSpec((rows, ncls), lambda i: (i, 0)),
        out_shape=jax.ShapeDtypeStruct((N, ncls), f32),
    )(pn, dcnt, b.reshape(1, HID), g.reshape(1, HID), be.reshape(1, HID),
      rm.reshape(1, HID), rv.reshape(1, HID), wc1, bc1.reshape(1, h1),
      wc2, bc2.reshape(1, ncls))


def kernel(x, hyperedge_index, W0, b0, g0, be0, rm0, rv0,
           W1, b1, g1, be1, rm1, rv1, W2, b2, g2, be2, rm2, rv2,
           Wc1, bc1, Wc2, bc2):
    pad = jnp.asarray(_PAD_IDX)

    def prep(idx):
        return jnp.concatenate([idx.reshape(NW, PER_W), pad], axis=1).reshape(
            NW, NCHUNK, CHUNK)

    p_src = prep(hyperedge_index[0])
    p_he = prep(hyperedge_index[1])
    zeros64 = jnp.zeros((MP, HID), f32)
    zeros16 = jnp.zeros((MP, 16), f32)
    ones16 = jnp.ones((CHUNK, 16), f32)

    dcnt, ecnt = _hist_call()(p_src, p_he, zeros16, ones16)
    stage1 = _stage1_call()

    params = [(b0, g0, be0, rm0, rv0),
              (b1, g1, be1, rm1, rv1),
              (b2, g2, be2, rm2, rv2)]
    next_w = [W1, W2]
    xw = _tc_matmul0(x, W0)
    for li, (b, g, be, rm, rv) in enumerate(params):
        pe = stage1(xw, p_src, p_he, zeros64)
        out_e = _tc_combine_e(pe, ecnt)
        pn = stage1(out_e, p_he, p_src, zeros64)
        if li < 2:
            xw = _tc_epilogue_matmul(pn, dcnt, b, g, be, rm, rv, next_w[li])
        else:
            return _tc_epilogue_head(pn, dcnt, b, g, be, rm, rv,
                                     Wc1, bc1, Wc2, bc2)


# speedup vs baseline: 1.1066x; 1.0551x over previous
import functools

import jax
import jax.numpy as jnp
import numpy as np
from jax import lax
from jax.experimental import pallas as pl
from jax.experimental.pallas import tpu as pltpu
from jax.experimental.pallas import tpu_sc as plsc

N = 10000
M = 10000
NNZ = 320000
HID = 64
EPS = 1e-5

NC = 2
NS = 16
NW = NC * NS
PER_W = NNZ // NW
CHUNK = 128
PER_W_PAD = 10240
PAD = PER_W_PAD - PER_W
NCHUNK = PER_W_PAD // CHUNK
NBUF = 4
MP = 10240
ROWS_PER_TILE = MP // NS
SUB = 128
NSUB = ROWS_PER_TILE // SUB

f32 = jnp.float32

_PAD_SCAT = (N + (np.arange(NW, dtype=np.int32)[:, None]
                  + np.arange(PAD, dtype=np.int32)[None, :]) % (MP - N))
_PAD_GATH = ((np.arange(NW, dtype=np.int32)[:, None] * 37
              + np.arange(PAD, dtype=np.int32)[None, :] * 41) % N)


def _mesh():
    return plsc.VectorSubcoreMesh(
        core_axis_name="c", subcore_axis_name="s", num_cores=NC, num_subcores=NS
    )


def _sc_compiler_params():
    return pltpu.CompilerParams(use_tc_tiling_on_sc=False)


def _ring_scratch():
    return ([pltpu.VMEM((NCHUNK, CHUNK), jnp.int32)] * 2
            + [pltpu.VMEM((CHUNK, HID), f32)] * NBUF
            + [pltpu.VMEM_SHARED((MP, HID), f32)]
            + [pltpu.SemaphoreType.DMA] * (2 * NBUF))


def _run_ring(table_hbm, gix_v, six_v, bufs, acc, gsems, ssems):
    def start_gather(ci, b):
        pltpu.async_copy(table_hbm.at[gix_v.at[ci]], bufs[b], gsems[b])

    def wait_gather(b):
        pltpu.make_async_copy(table_hbm.at[gix_v.at[0]], bufs[b],
                              gsems[b]).wait()

    def start_scatter(ci, b):
        pltpu.async_copy(bufs[b], acc.at[six_v.at[ci]], ssems[b], add=True)

    def wait_scatter(b):
        pltpu.make_async_copy(bufs[b], acc.at[six_v.at[0]], ssems[b]).wait()

    for b in range(NBUF):
        start_gather(b, b)

    @pl.loop(0, NCHUNK, step=NBUF)
    def _(ci):
        for b in range(NBUF):
            wait_gather(b)
            start_scatter(ci + b, b)
        for b in range(NBUF):
            wait_scatter(b)
            start_gather(jnp.minimum(ci + NBUF + b, NCHUNK - 1), b)

    for b in range(NBUF):
        wait_gather(b)


@functools.cache
def _stage1_call():
    @functools.partial(
        pl.kernel,
        out_type=jax.ShapeDtypeStruct((NC, MP, HID), f32),
        mesh=_mesh(),
        compiler_params=_sc_compiler_params(),
        scratch_types=_ring_scratch(),
    )
    def stage1(table_hbm, gidx_hbm, sidx_hbm, zeros_hbm, out_hbm,
               gix_v, six_v, *rest):
        bufs = rest[:NBUF]
        acc = rest[NBUF]
        gsems = rest[NBUF + 1:NBUF + 1 + NBUF]
        ssems = rest[NBUF + 1 + NBUF:]
        cid = lax.axis_index("c")
        sid = lax.axis_index("s")
        wid = cid * NS + sid
        base = sid * ROWS_PER_TILE
        pltpu.sync_copy(zeros_hbm.at[pl.ds(base, ROWS_PER_TILE)],
                        acc.at[pl.ds(base, ROWS_PER_TILE)])
        pltpu.sync_copy(gidx_hbm.at[wid], gix_v)
        pltpu.sync_copy(sidx_hbm.at[wid], six_v)
        plsc.subcore_barrier()
        _run_ring(table_hbm, gix_v, six_v, bufs, acc, gsems, ssems)
        plsc.subcore_barrier()
        pltpu.sync_copy(acc.at[pl.ds(base, ROWS_PER_TILE)],
                        out_hbm.at[cid, pl.ds(base, ROWS_PER_TILE)])

    return stage1


@functools.cache
def _hist_call():
    @functools.partial(
        pl.kernel,
        out_type=[jax.ShapeDtypeStruct((NC, MP, 16), f32),
                  jax.ShapeDtypeStruct((NC, MP, 16), f32)],
        mesh=_mesh(),
        compiler_params=_sc_compiler_params(),
        scratch_types=[
            pltpu.VMEM((NCHUNK, CHUNK), jnp.int32),
            pltpu.VMEM((NCHUNK, CHUNK), jnp.int32),
            pltpu.VMEM((CHUNK, 16), f32),
            pltpu.VMEM_SHARED((MP, 16), f32),
            pltpu.VMEM_SHARED((MP, 16), f32),
            pltpu.SemaphoreType.DMA,
            pltpu.SemaphoreType.DMA,
        ],
    )
    def hist(src_hbm, he_hbm, zeros16_hbm, ones_hbm, dout_hbm, eout_hbm,
             src_v, he_v, ones_v, dacc, eacc, sem_d, sem_e):
        cid = lax.axis_index("c")
        sid = lax.axis_index("s")
        wid = cid * NS + sid
        base = sid * ROWS_PER_TILE
        pltpu.sync_copy(zeros16_hbm.at[pl.ds(base, ROWS_PER_TILE)],
                        dacc.at[pl.ds(base, ROWS_PER_TILE)])
        pltpu.sync_copy(zeros16_hbm.at[pl.ds(base, ROWS_PER_TILE)],
                        eacc.at[pl.ds(base, ROWS_PER_TILE)])
        pltpu.sync_copy(ones_hbm, ones_v)
        pltpu.sync_copy(src_hbm.at[wid], src_v)
        pltpu.sync_copy(he_hbm.at[wid], he_v)
        plsc.subcore_barrier()

        K = 5

        @pl.loop(0, NCHUNK, step=K)
        def _(ci):
            for j in range(K):
                pltpu.async_copy(ones_v, dacc.at[src_v.at[ci + j]], sem_d,
                                 add=True)
                pltpu.async_copy(ones_v, eacc.at[he_v.at[ci + j]], sem_e,
                                 add=True)
            for j in range(K):
                pltpu.make_async_copy(ones_v, dacc.at[src_v.at[ci]],
                                      sem_d).wait()
                pltpu.make_async_copy(ones_v, eacc.at[he_v.at[ci]],
                                      sem_e).wait()

        plsc.subcore_barrier()
        pltpu.sync_copy(dacc.at[pl.ds(base, ROWS_PER_TILE)],
                        dout_hbm.at[cid, pl.ds(base, ROWS_PER_TILE)])
        pltpu.sync_copy(eacc.at[pl.ds(base, ROWS_PER_TILE)],
                        eout_hbm.at[cid, pl.ds(base, ROWS_PER_TILE)])

    return hist


BN_ROWS = 1024


def _tc_combine_e(pe, ecnt):
    def body(pe_ref, cnt_ref, o_ref):
        s = pe_ref[0] + pe_ref[1]
        edeg = cnt_ref[0, :, 0:1] + cnt_ref[1, :, 0:1]
        binv = jnp.where(edeg > 0, 1.0 / edeg, 0.0)
        o_ref[...] = s * binv

    return pl.pallas_call(
        body,
        grid=(MP // BN_ROWS,),
        in_specs=[
            pl.BlockSpec((NC, BN_ROWS, HID), lambda i: (0, i, 0)),
            pl.BlockSpec((NC, BN_ROWS, 16), lambda i: (0, i, 0)),
        ],
        out_specs=pl.BlockSpec((BN_ROWS, HID), lambda i: (i, 0)),
        out_shape=jax.ShapeDtypeStruct((MP, HID), f32),
    )(pe, ecnt)


def _tc_matmul0(x, w):
    def body(x_ref, w_ref, o_ref):
        o_ref[...] = jnp.dot(x_ref[...], w_ref[...],
                             preferred_element_type=f32)

    d_in = x.shape[1]
    return pl.pallas_call(
        body,
        grid=(MP // BN_ROWS,),
        in_specs=[
            pl.BlockSpec((BN_ROWS, d_in), lambda i: (i, 0)),
            pl.BlockSpec((d_in, HID), lambda i: (0, 0)),
        ],
        out_specs=pl.BlockSpec((BN_ROWS, HID), lambda i: (i, 0)),
        out_shape=jax.ShapeDtypeStruct((MP, HID), f32),
    )(x, w)


def _tc_epilogue_matmul(pn, dcnt, b, g, be, rm, rv, w):
    def body(pn_ref, cnt_ref, b_ref, g_ref, be_ref, rm_ref, rv_ref, w_ref,
             o_ref):
        s = pn_ref[0] + pn_ref[1]
        deg = cnt_ref[0, :, 0:1] + cnt_ref[1, :, 0:1]
        dinv = jnp.where(deg > 0, 1.0 / deg, 0.0)
        scale = g_ref[...] * lax.rsqrt(rv_ref[...] + EPS)
        shift = (b_ref[...] - rm_ref[...]) * scale + be_ref[...]
        h = jnp.maximum(s * dinv * scale + shift, 0.0)
        o_ref[...] = jnp.dot(h, w_ref[...], preferred_element_type=f32)

    vec = lambda: pl.BlockSpec((1, HID), lambda i: (0, 0))
    return pl.pallas_call(
        body,
        grid=(MP // BN_ROWS,),
        in_specs=[
            pl.BlockSpec((NC, BN_ROWS, HID), lambda i: (0, i, 0)),
            pl.BlockSpec((NC, BN_ROWS, 16), lambda i: (0, i, 0)),
            vec(), vec(), vec(), vec(), vec(),
            pl.BlockSpec((HID, HID), lambda i: (0, 0)),
        ],
        out_specs=pl.BlockSpec((BN_ROWS, HID), lambda i: (i, 0)),
        out_shape=jax.ShapeDtypeStruct((MP, HID), f32),
    )(pn, dcnt, b.reshape(1, HID), g.reshape(1, HID), be.reshape(1, HID),
      rm.reshape(1, HID), rv.reshape(1, HID), w)


def _tc_epilogue_head(pn, dcnt, b, g, be, rm, rv, wc1, bc1, wc2, bc2):
    h1 = wc1.shape[1]
    ncls = wc2.shape[1]
    rows = 1000

    def body(pn_ref, cnt_ref, b_ref, g_ref, be_ref, rm_ref, rv_ref,
             wc1_ref, bc1_ref, wc2_ref, bc2_ref, o_ref):
        s = pn_ref[0] + pn_ref[1]
        deg = cnt_ref[0, :, 0:1] + cnt_ref[1, :, 0:1]
        dinv = jnp.where(deg > 0, 1.0 / deg, 0.0)
        scale = g_ref[...] * lax.rsqrt(rv_ref[...] + EPS)
        shift = (b_ref[...] - rm_ref[...]) * scale + be_ref[...]
        h = jnp.maximum(s * dinv * scale + shift, 0.0)
        t = jnp.maximum(
            jnp.dot(h, wc1_ref[...], preferred_element_type=f32)
            + bc1_ref[...], 0.0)
        o_ref[...] = (jnp.dot(t, wc2_ref[...], preferred_element_type=f32)
                      + bc2_ref[...])

    vec = lambda: pl.BlockSpec((1, HID), lambda i: (0, 0))
    return pl.pallas_call(
        body,
        grid=(N // rows,),
        in_specs=[
            pl.BlockSpec((NC, rows, HID), lambda i: (0, i, 0)),
            pl.BlockSpec((NC, rows, 16), lambda i: (0, i, 0)),
            vec(), vec(), vec(), vec(), vec(),
            pl.BlockSpec((HID, h1), lambda i: (0, 0)),
            pl.BlockSpec((1, h1), lambda i: (0, 0)),
            pl.BlockSpec((h1, ncls), lambda i: (0, 0)),
            pl.BlockSpec((1, ncls), lambda i: (0, 0)),
        ],
        out_specs=pl.BlockSpec((rows, ncls), lambda i: (i, 0)),
        out_shape=jax.ShapeDtypeStruct((N, ncls), f32),
    )(pn, dcnt, b.reshape(1, HID), g.reshape(1, HID), be.reshape(1, HID),
      rm.reshape(1, HID), rv.reshape(1, HID), wc1, bc1.reshape(1, h1),
      wc2, bc2.reshape(1, ncls))


def kernel(x, hyperedge_index, W0, b0, g0, be0, rm0, rv0,
           W1, b1, g1, be1, rm1, rv1, W2, b2, g2, be2, rm2, rv2,
           Wc1, bc1, Wc2, bc2):
    pad_s = jnp.asarray(_PAD_SCAT)
    pad_g = jnp.asarray(_PAD_GATH)

    def prep(idx, pad):
        return jnp.concatenate([idx.reshape(NW, PER_W), pad], axis=1).reshape(
            NW, NCHUNK, CHUNK)

    s_src = prep(hyperedge_index[0], pad_s)
    g_src = prep(hyperedge_index[0], pad_g)
    s_he = prep(hyperedge_index[1], pad_s)
    g_he = prep(hyperedge_index[1], pad_g)
    zeros64 = jnp.zeros((MP, HID), f32)
    zeros16 = jnp.zeros((MP, 16), f32)
    ones16 = jnp.ones((CHUNK, 16), f32)

    dcnt, ecnt = _hist_call()(s_src, s_he, zeros16, ones16)
    stage1 = _stage1_call()

    params = [(b0, g0, be0, rm0, rv0),
              (b1, g1, be1, rm1, rv1),
              (b2, g2, be2, rm2, rv2)]
    next_w = [W1, W2]
    xw = _tc_matmul0(x, W0)
    for li, (b, g, be, rm, rv) in enumerate(params):
        pe = stage1(xw, g_src, s_he, zeros64)
        out_e = _tc_combine_e(pe, ecnt)
        pn = stage1(out_e, g_he, s_src, zeros64)
        if li < 2:
            xw = _tc_epilogue_matmul(pn, dcnt, b, g, be, rm, rv, next_w[li])
        else:
            return _tc_epilogue_head(pn, dcnt, b, g, be, rm, rv,
                                     Wc1, bc1, Wc2, bc2)


# speedup vs baseline: 1.1263x; 1.0178x over previous
import functools

import jax
import jax.numpy as jnp
import numpy as np
from jax import lax
from jax.experimental import pallas as pl
from jax.experimental.pallas import tpu as pltpu
from jax.experimental.pallas import tpu_sc as plsc

N = 10000
M = 10000
NNZ = 320000
HID = 64
EPS = 1e-5

NC = 2
NS = 16
NW = NC * NS
PER_W = NNZ // NW
CHUNK = 128
PER_W_PAD = 10240
PAD = PER_W_PAD - PER_W
NCHUNK = PER_W_PAD // CHUNK
NBUF = 8
MP = 10240
ROWS_PER_TILE = MP // NS
SUB = 128
NSUB = ROWS_PER_TILE // SUB

f32 = jnp.float32

_PAD_SCAT = (N + (np.arange(NW, dtype=np.int32)[:, None]
                  + np.arange(PAD, dtype=np.int32)[None, :]) % (MP - N))
_PAD_GATH = ((np.arange(NW, dtype=np.int32)[:, None] * 37
              + np.arange(PAD, dtype=np.int32)[None, :] * 41) % N)


def _mesh():
    return plsc.VectorSubcoreMesh(
        core_axis_name="c", subcore_axis_name="s", num_cores=NC, num_subcores=NS
    )


def _sc_compiler_params():
    return pltpu.CompilerParams(use_tc_tiling_on_sc=False)


def _ring_scratch():
    return ([pltpu.VMEM((NCHUNK, CHUNK), jnp.int32)] * 2
            + [pltpu.VMEM((CHUNK, HID), f32)] * NBUF
            + [pltpu.VMEM_SHARED((MP, HID), f32)]
            + [pltpu.SemaphoreType.DMA] * (2 * NBUF))


def _run_ring(table_hbm, gix_v, six_v, bufs, acc, gsems, ssems):
    def start_gather(ci, b):
        pltpu.async_copy(table_hbm.at[gix_v.at[ci]], bufs[b], gsems[b])

    def wait_gather(b):
        pltpu.make_async_copy(table_hbm.at[gix_v.at[0]], bufs[b],
                              gsems[b]).wait()

    def start_scatter(ci, b):
        pltpu.async_copy(bufs[b], acc.at[six_v.at[ci]], ssems[b], add=True)

    def wait_scatter(b):
        pltpu.make_async_copy(bufs[b], acc.at[six_v.at[0]], ssems[b]).wait()

    for b in range(NBUF):
        start_gather(b, b)

    @pl.loop(0, NCHUNK, step=NBUF)
    def _(ci):
        for b in range(NBUF):
            wait_gather(b)
            start_scatter(ci + b, b)
        for b in range(NBUF):
            wait_scatter(b)
            start_gather(jnp.minimum(ci + NBUF + b, NCHUNK - 1), b)

    for b in range(NBUF):
        wait_gather(b)


@functools.cache
def _stage1_call():
    @functools.partial(
        pl.kernel,
        out_type=jax.ShapeDtypeStruct((NC, MP, HID), f32),
        mesh=_mesh(),
        compiler_params=_sc_compiler_params(),
        scratch_types=_ring_scratch(),
    )
    def stage1(table_hbm, gidx_hbm, sidx_hbm, zeros_hbm, out_hbm,
               gix_v, six_v, *rest):
        bufs = rest[:NBUF]
        acc = rest[NBUF]
        gsems = rest[NBUF + 1:NBUF + 1 + NBUF]
        ssems = rest[NBUF + 1 + NBUF:]
        cid = lax.axis_index("c")
        sid = lax.axis_index("s")
        wid = cid * NS + sid
        base = sid * ROWS_PER_TILE
        pltpu.sync_copy(zeros_hbm.at[pl.ds(base, ROWS_PER_TILE)],
                        acc.at[pl.ds(base, ROWS_PER_TILE)])
        pltpu.sync_copy(gidx_hbm.at[wid], gix_v)
        pltpu.sync_copy(sidx_hbm.at[wid], six_v)
        plsc.subcore_barrier()
        _run_ring(table_hbm, gix_v, six_v, bufs, acc, gsems, ssems)
        plsc.subcore_barrier()
        pltpu.sync_copy(acc.at[pl.ds(base, ROWS_PER_TILE)],
                        out_hbm.at[cid, pl.ds(base, ROWS_PER_TILE)])

    return stage1


@functools.cache
def _hist_call():
    @functools.partial(
        pl.kernel,
        out_type=[jax.ShapeDtypeStruct((NC, MP, 16), f32),
                  jax.ShapeDtypeStruct((NC, MP, 16), f32)],
        mesh=_mesh(),
        compiler_params=_sc_compiler_params(),
        scratch_types=[
            pltpu.VMEM((NCHUNK, CHUNK), jnp.int32),
            pltpu.VMEM((NCHUNK, CHUNK), jnp.int32),
            pltpu.VMEM((CHUNK, 16), f32),
            pltpu.VMEM_SHARED((MP, 16), f32),
            pltpu.VMEM_SHARED((MP, 16), f32),
            pltpu.SemaphoreType.DMA,
            pltpu.SemaphoreType.DMA,
        ],
    )
    def hist(src_hbm, he_hbm, zeros16_hbm, ones_hbm, dout_hbm, eout_hbm,
             src_v, he_v, ones_v, dacc, eacc, sem_d, sem_e):
        cid = lax.axis_index("c")
        sid = lax.axis_index("s")
        wid = cid * NS + sid
        base = sid * ROWS_PER_TILE
        pltpu.sync_copy(zeros16_hbm.at[pl.ds(base, ROWS_PER_TILE)],
                        dacc.at[pl.ds(base, ROWS_PER_TILE)])
        pltpu.sync_copy(zeros16_hbm.at[pl.ds(base, ROWS_PER_TILE)],
                        eacc.at[pl.ds(base, ROWS_PER_TILE)])
        pltpu.sync_copy(ones_hbm, ones_v)
        pltpu.sync_copy(src_hbm.at[wid], src_v)
        pltpu.sync_copy(he_hbm.at[wid], he_v)
        plsc.subcore_barrier()

        K = 5

        @pl.loop(0, NCHUNK, step=K)
        def _(ci):
            for j in range(K):
                pltpu.async_copy(ones_v, dacc.at[src_v.at[ci + j]], sem_d,
                                 add=True)
                pltpu.async_copy(ones_v, eacc.at[he_v.at[ci + j]], sem_e,
                                 add=True)
            for j in range(K):
                pltpu.make_async_copy(ones_v, dacc.at[src_v.at[ci]],
                                      sem_d).wait()
                pltpu.make_async_copy(ones_v, eacc.at[he_v.at[ci]],
                                      sem_e).wait()

        plsc.subcore_barrier()
        pltpu.sync_copy(dacc.at[pl.ds(base, ROWS_PER_TILE)],
                        dout_hbm.at[cid, pl.ds(base, ROWS_PER_TILE)])
        pltpu.sync_copy(eacc.at[pl.ds(base, ROWS_PER_TILE)],
                        eout_hbm.at[cid, pl.ds(base, ROWS_PER_TILE)])

    return hist


BN_ROWS = 1024


def _tc_combine_e(pe, ecnt):
    def body(pe_ref, cnt_ref, o_ref):
        s = pe_ref[0] + pe_ref[1]
        edeg = cnt_ref[0, :, 0:1] + cnt_ref[1, :, 0:1]
        binv = jnp.where(edeg > 0, 1.0 / edeg, 0.0)
        o_ref[...] = s * binv

    return pl.pallas_call(
        body,
        grid=(MP // BN_ROWS,),
        in_specs=[
            pl.BlockSpec((NC, BN_ROWS, HID), lambda i: (0, i, 0)),
            pl.BlockSpec((NC, BN_ROWS, 16), lambda i: (0, i, 0)),
        ],
        out_specs=pl.BlockSpec((BN_ROWS, HID), lambda i: (i, 0)),
        out_shape=jax.ShapeDtypeStruct((MP, HID), f32),
    )(pe, ecnt)


def _tc_matmul0(x, w):
    def body(x_ref, w_ref, o_ref):
        o_ref[...] = jnp.dot(x_ref[...], w_ref[...],
                             preferred_element_type=f32)

    d_in = x.shape[1]
    return pl.pallas_call(
        body,
        grid=(MP // BN_ROWS,),
        in_specs=[
            pl.BlockSpec((BN_ROWS, d_in), lambda i: (i, 0)),
            pl.BlockSpec((d_in, HID), lambda i: (0, 0)),
        ],
        out_specs=pl.BlockSpec((BN_ROWS, HID), lambda i: (i, 0)),
        out_shape=jax.ShapeDtypeStruct((MP, HID), f32),
    )(x, w)


def _tc_epilogue_matmul(pn, dcnt, b, g, be, rm, rv, w):
    def body(pn_ref, cnt_ref, b_ref, g_ref, be_ref, rm_ref, rv_ref, w_ref,
             o_ref):
        s = pn_ref[0] + pn_ref[1]
        deg = cnt_ref[0, :, 0:1] + cnt_ref[1, :, 0:1]
        dinv = jnp.where(deg > 0, 1.0 / deg, 0.0)
        scale = g_ref[...] * lax.rsqrt(rv_ref[...] + EPS)
        shift = (b_ref[...] - rm_ref[...]) * scale + be_ref[...]
        h = jnp.maximum(s * dinv * scale + shift, 0.0)
        o_ref[...] = jnp.dot(h, w_ref[...], preferred_element_type=f32)

    vec = lambda: pl.BlockSpec((1, HID), lambda i: (0, 0))
    return pl.pallas_call(
        body,
        grid=(MP // BN_ROWS,),
        in_specs=[
            pl.BlockSpec((NC, BN_ROWS, HID), lambda i: (0, i, 0)),
            pl.BlockSpec((NC, BN_ROWS, 16), lambda i: (0, i, 0)),
            vec(), vec(), vec(), vec(), vec(),
            pl.BlockSpec((HID, HID), lambda i: (0, 0)),
        ],
        out_specs=pl.BlockSpec((BN_ROWS, HID), lambda i: (i, 0)),
        out_shape=jax.ShapeDtypeStruct((MP, HID), f32),
    )(pn, dcnt, b.reshape(1, HID), g.reshape(1, HID), be.reshape(1, HID),
      rm.reshape(1, HID), rv.reshape(1, HID), w)


def _tc_epilogue_head(pn, dcnt, b, g, be, rm, rv, wc1, bc1, wc2, bc2):
    h1 = wc1.shape[1]
    ncls = wc2.shape[1]
    rows = 1000

    def body(pn_ref, cnt_ref, b_ref, g_ref, be_ref, rm_ref, rv_ref,
             wc1_ref, bc1_ref, wc2_ref, bc2_ref, o_ref):
        s = pn_ref[0] + pn_ref[1]
        deg = cnt_ref[0, :, 0:1] + cnt_ref[1, :, 0:1]
        dinv = jnp.where(deg > 0, 1.0 / deg, 0.0)
        scale = g_ref[...] * lax.rsqrt(rv_ref[...] + EPS)
        shift = (b_ref[...] - rm_ref[...]) * scale + be_ref[...]
        h = jnp.maximum(s * dinv * scale + shift, 0.0)
        t = jnp.maximum(
            jnp.dot(h, wc1_ref[...], preferred_element_type=f32)
            + bc1_ref[...], 0.0)
        o_ref[...] = (jnp.dot(t, wc2_ref[...], preferred_element_type=f32)
                      + bc2_ref[...])

    vec = lambda: pl.BlockSpec((1, HID), lambda i: (0, 0))
    return pl.pallas_call(
        body,
        grid=(N // rows,),
        in_specs=[
            pl.BlockSpec((NC, rows, HID), lambda i: (0, i, 0)),
            pl.BlockSpec((NC, rows, 16), lambda i: (0, i, 0)),
            vec(), vec(), vec(), vec(), vec(),
            pl.BlockSpec((HID, h1), lambda i: (0, 0)),
            pl.BlockSpec((1, h1), lambda i: (0, 0)),
            pl.BlockSpec((h1, ncls), lambda i: (0, 0)),
            pl.BlockSpec((1, ncls), lambda i: (0, 0)),
        ],
        out_specs=pl.BlockSpec((rows, ncls), lambda i: (i, 0)),
        out_shape=jax.ShapeDtypeStruct((N, ncls), f32),
    )(pn, dcnt, b.reshape(1, HID), g.reshape(1, HID), be.reshape(1, HID),
      rm.reshape(1, HID), rv.reshape(1, HID), wc1, bc1.reshape(1, h1),
      wc2, bc2.reshape(1, ncls))


def kernel(x, hyperedge_index, W0, b0, g0, be0, rm0, rv0,
           W1, b1, g1, be1, rm1, rv1, W2, b2, g2, be2, rm2, rv2,
           Wc1, bc1, Wc2, bc2):
    pad_s = jnp.asarray(_PAD_SCAT)
    pad_g = jnp.asarray(_PAD_GATH)

    def prep(idx, pad):
        return jnp.concatenate([idx.reshape(NW, PER_W), pad], axis=1).reshape(
            NW, NCHUNK, CHUNK)

    s_src = prep(hyperedge_index[0], pad_s)
    g_src = prep(hyperedge_index[0], pad_g)
    s_he = prep(hyperedge_index[1], pad_s)
    g_he = prep(hyperedge_index[1], pad_g)
    zeros64 = jnp.zeros((MP, HID), f32)
    zeros16 = jnp.zeros((MP, 16), f32)
    ones16 = jnp.ones((CHUNK, 16), f32)

    dcnt, ecnt = _hist_call()(s_src, s_he, zeros16, ones16)
    stage1 = _stage1_call()

    params = [(b0, g0, be0, rm0, rv0),
              (b1, g1, be1, rm1, rv1),
              (b2, g2, be2, rm2, rv2)]
    next_w = [W1, W2]
    xw = _tc_matmul0(x, W0)
    for li, (b, g, be, rm, rv) in enumerate(params):
        pe = stage1(xw, g_src, s_he, zeros64)
        out_e = _tc_combine_e(pe, ecnt)
        pn = stage1(out_e, g_he, s_src, zeros64)
        if li < 2:
            xw = _tc_epilogue_matmul(pn, dcnt, b, g, be, rm, rv, next_w[li])
        else:
            return _tc_epilogue_head(pn, dcnt, b, g, be, rm, rv,
                                     Wc1, bc1, Wc2, bc2)


# speedup vs baseline: 1.1511x; 1.0220x over previous
import functools

import jax
import jax.numpy as jnp
import numpy as np
from jax import lax
from jax.experimental import pallas as pl
from jax.experimental.pallas import tpu as pltpu
from jax.experimental.pallas import tpu_sc as plsc

N = 10000
M = 10000
NNZ = 320000
HID = 64
EPS = 1e-5

NC = 2
NS = 16
NW = NC * NS
PER_W = NNZ // NW
CHUNK = 128
PER_W_PAD = 10240
PAD = PER_W_PAD - PER_W
NCHUNK = PER_W_PAD // CHUNK
NBUF = 8
MP = 10240
ROWS_PER_TILE = MP // NS
SUB = 128
NSUB = ROWS_PER_TILE // SUB

f32 = jnp.float32

_PAD_SCAT = (N + (np.arange(NW, dtype=np.int32)[:, None]
                  + np.arange(PAD, dtype=np.int32)[None, :]) % (MP - N))
_PAD_GATH = ((np.arange(NW, dtype=np.int32)[:, None] * 37
              + np.arange(PAD, dtype=np.int32)[None, :] * 41) % N)


def _mesh():
    return plsc.VectorSubcoreMesh(
        core_axis_name="c", subcore_axis_name="s", num_cores=NC, num_subcores=NS
    )


def _sc_compiler_params():
    return pltpu.CompilerParams(use_tc_tiling_on_sc=False)


def _ring_scratch():
    return ([pltpu.VMEM((NCHUNK, CHUNK), jnp.int32)] * 2
            + [pltpu.VMEM((CHUNK, HID), f32)] * NBUF
            + [pltpu.VMEM_SHARED((MP, HID), f32)]
            + [pltpu.SemaphoreType.DMA] * (2 * NBUF))


def _run_ring(table_hbm, gix_v, six_v, bufs, acc, gsems, ssems):
    def start_gather(ci, b):
        pltpu.async_copy(table_hbm.at[gix_v.at[ci]], bufs[b], gsems[b])

    def wait_gather(b):
        pltpu.make_async_copy(table_hbm.at[gix_v.at[0]], bufs[b],
                              gsems[b]).wait()

    def start_scatter(ci, b):
        pltpu.async_copy(bufs[b], acc.at[six_v.at[ci]], ssems[b], add=True)

    def wait_scatter(b):
        pltpu.make_async_copy(bufs[b], acc.at[six_v.at[0]], ssems[b]).wait()

    for b in range(NBUF):
        start_gather(b, b)

    @pl.loop(0, NCHUNK, step=NBUF)
    def _(ci):
        for b in range(NBUF):
            wait_gather(b)
            start_scatter(ci + b, b)
        for b in range(NBUF):
            wait_scatter(b)
            start_gather(jnp.minimum(ci + NBUF + b, NCHUNK - 1), b)

    for b in range(NBUF):
        wait_gather(b)


@functools.cache
def _stage1_call():
    @functools.partial(
        pl.kernel,
        out_type=jax.ShapeDtypeStruct((NC, MP, HID), f32),
        mesh=_mesh(),
        compiler_params=_sc_compiler_params(),
        scratch_types=_ring_scratch(),
    )
    def stage1(table_hbm, gidx_hbm, sidx_hbm, zeros_hbm, out_hbm,
               gix_v, six_v, *rest):
        bufs = rest[:NBUF]
        acc = rest[NBUF]
        gsems = rest[NBUF + 1:NBUF + 1 + NBUF]
        ssems = rest[NBUF + 1 + NBUF:]
        cid = lax.axis_index("c")
        sid = lax.axis_index("s")
        wid = cid * NS + sid
        base = sid * ROWS_PER_TILE
        d0 = pltpu.async_copy(zeros_hbm.at[pl.ds(base, ROWS_PER_TILE)],
                              acc.at[pl.ds(base, ROWS_PER_TILE)], gsems[0])
        d1 = pltpu.async_copy(gidx_hbm.at[wid], gix_v, gsems[1])
        d2 = pltpu.async_copy(sidx_hbm.at[wid], six_v, gsems[2])
        d0.wait()
        d1.wait()
        d2.wait()
        plsc.subcore_barrier()
        _run_ring(table_hbm, gix_v, six_v, bufs, acc, gsems, ssems)
        plsc.subcore_barrier()
        pltpu.sync_copy(acc.at[pl.ds(base, ROWS_PER_TILE)],
                        out_hbm.at[cid, pl.ds(base, ROWS_PER_TILE)])

    return stage1


@functools.cache
def _hist_call():
    @functools.partial(
        pl.kernel,
        out_type=[jax.ShapeDtypeStruct((NC, MP, 16), f32),
                  jax.ShapeDtypeStruct((NC, MP, 16), f32)],
        mesh=_mesh(),
        compiler_params=_sc_compiler_params(),
        scratch_types=[
            pltpu.VMEM((NCHUNK, CHUNK), jnp.int32),
            pltpu.VMEM((NCHUNK, CHUNK), jnp.int32),
            pltpu.VMEM((CHUNK, 16), f32),
            pltpu.VMEM_SHARED((MP, 16), f32),
            pltpu.VMEM_SHARED((MP, 16), f32),
            pltpu.SemaphoreType.DMA,
            pltpu.SemaphoreType.DMA,
        ],
    )
    def hist(src_hbm, he_hbm, zeros16_hbm, ones_hbm, dout_hbm, eout_hbm,
             src_v, he_v, ones_v, dacc, eacc, sem_d, sem_e):
        cid = lax.axis_index("c")
        sid = lax.axis_index("s")
        wid = cid * NS + sid
        base = sid * ROWS_PER_TILE
        d0 = pltpu.async_copy(zeros16_hbm.at[pl.ds(base, ROWS_PER_TILE)],
                              dacc.at[pl.ds(base, ROWS_PER_TILE)], sem_d)
        d1 = pltpu.async_copy(zeros16_hbm.at[pl.ds(base, ROWS_PER_TILE)],
                              eacc.at[pl.ds(base, ROWS_PER_TILE)], sem_e)
        pltpu.sync_copy(ones_hbm, ones_v)
        pltpu.sync_copy(src_hbm.at[wid], src_v)
        pltpu.sync_copy(he_hbm.at[wid], he_v)
        d0.wait()
        d1.wait()
        plsc.subcore_barrier()

        K = 5

        @pl.loop(0, NCHUNK, step=K)
        def _(ci):
            for j in range(K):
                pltpu.async_copy(ones_v, dacc.at[src_v.at[ci + j]], sem_d,
                                 add=True)
                pltpu.async_copy(ones_v, eacc.at[he_v.at[ci + j]], sem_e,
                                 add=True)
            for j in range(K):
                pltpu.make_async_copy(ones_v, dacc.at[src_v.at[ci]],
                                      sem_d).wait()
                pltpu.make_async_copy(ones_v, eacc.at[he_v.at[ci]],
                                      sem_e).wait()

        plsc.subcore_barrier()
        pltpu.sync_copy(dacc.at[pl.ds(base, ROWS_PER_TILE)],
                        dout_hbm.at[cid, pl.ds(base, ROWS_PER_TILE)])
        pltpu.sync_copy(eacc.at[pl.ds(base, ROWS_PER_TILE)],
                        eout_hbm.at[cid, pl.ds(base, ROWS_PER_TILE)])

    return hist


BN_ROWS = 1024


def _tc_combine_e(pe, ecnt):
    def body(pe_ref, cnt_ref, o_ref):
        s = pe_ref[0] + pe_ref[1]
        edeg = cnt_ref[0, :, 0:1] + cnt_ref[1, :, 0:1]
        binv = jnp.where(edeg > 0, 1.0 / edeg, 0.0)
        o_ref[...] = s * binv

    return pl.pallas_call(
        body,
        grid=(MP // BN_ROWS,),
        in_specs=[
            pl.BlockSpec((NC, BN_ROWS, HID), lambda i: (0, i, 0)),
            pl.BlockSpec((NC, BN_ROWS, 16), lambda i: (0, i, 0)),
        ],
        out_specs=pl.BlockSpec((BN_ROWS, HID), lambda i: (i, 0)),
        out_shape=jax.ShapeDtypeStruct((MP, HID), f32),
    )(pe, ecnt)


def _tc_matmul0(x, w):
    def body(x_ref, w_ref, o_ref):
        o_ref[...] = jnp.dot(x_ref[...], w_ref[...],
                             preferred_element_type=f32)

    d_in = x.shape[1]
    return pl.pallas_call(
        body,
        grid=(MP // BN_ROWS,),
        in_specs=[
            pl.BlockSpec((BN_ROWS, d_in), lambda i: (i, 0)),
            pl.BlockSpec((d_in, HID), lambda i: (0, 0)),
        ],
        out_specs=pl.BlockSpec((BN_ROWS, HID), lambda i: (i, 0)),
        out_shape=jax.ShapeDtypeStruct((MP, HID), f32),
    )(x, w)


def _tc_epilogue_matmul(pn, dcnt, b, g, be, rm, rv, w):
    def body(pn_ref, cnt_ref, b_ref, g_ref, be_ref, rm_ref, rv_ref, w_ref,
             o_ref):
        s = pn_ref[0] + pn_ref[1]
        deg = cnt_ref[0, :, 0:1] + cnt_ref[1, :, 0:1]
        dinv = jnp.where(deg > 0, 1.0 / deg, 0.0)
        scale = g_ref[...] * lax.rsqrt(rv_ref[...] + EPS)
        shift = (b_ref[...] - rm_ref[...]) * scale + be_ref[...]
        h = jnp.maximum(s * dinv * scale + shift, 0.0)
        o_ref[...] = jnp.dot(h, w_ref[...], preferred_element_type=f32)

    vec = lambda: pl.BlockSpec((1, HID), lambda i: (0, 0))
    return pl.pallas_call(
        body,
        grid=(MP // BN_ROWS,),
        in_specs=[
            pl.BlockSpec((NC, BN_ROWS, HID), lambda i: (0, i, 0)),
            pl.BlockSpec((NC, BN_ROWS, 16), lambda i: (0, i, 0)),
            vec(), vec(), vec(), vec(), vec(),
            pl.BlockSpec((HID, HID), lambda i: (0, 0)),
        ],
        out_specs=pl.BlockSpec((BN_ROWS, HID), lambda i: (i, 0)),
        out_shape=jax.ShapeDtypeStruct((MP, HID), f32),
    )(pn, dcnt, b.reshape(1, HID), g.reshape(1, HID), be.reshape(1, HID),
      rm.reshape(1, HID), rv.reshape(1, HID), w)


def _tc_epilogue_head(pn, dcnt, b, g, be, rm, rv, wc1, bc1, wc2, bc2):
    h1 = wc1.shape[1]
    ncls = wc2.shape[1]
    rows = 1000

    def body(pn_ref, cnt_ref, b_ref, g_ref, be_ref, rm_ref, rv_ref,
             wc1_ref, bc1_ref, wc2_ref, bc2_ref, o_ref):
        s = pn_ref[0] + pn_ref[1]
        deg = cnt_ref[0, :, 0:1] + cnt_ref[1, :, 0:1]
        dinv = jnp.where(deg > 0, 1.0 / deg, 0.0)
        scale = g_ref[...] * lax.rsqrt(rv_ref[...] + EPS)
        shift = (b_ref[...] - rm_ref[...]) * scale + be_ref[...]
        h = jnp.maximum(s * dinv * scale + shift, 0.0)
        t = jnp.maximum(
            jnp.dot(h, wc1_ref[...], preferred_element_type=f32)
            + bc1_ref[...], 0.0)
        o_ref[...] = (jnp.dot(t, wc2_ref[...], preferred_element_type=f32)
                      + bc2_ref[...])

    vec = lambda: pl.BlockSpec((1, HID), lambda i: (0, 0))
    return pl.pallas_call(
        body,
        grid=(N // rows,),
        in_specs=[
            pl.BlockSpec((NC, rows, HID), lambda i: (0, i, 0)),
            pl.BlockSpec((NC, rows, 16), lambda i: (0, i, 0)),
            vec(), vec(), vec(), vec(), vec(),
            pl.BlockSpec((HID, h1), lambda i: (0, 0)),
            pl.BlockSpec((1, h1), lambda i: (0, 0)),
            pl.BlockSpec((h1, ncls), lambda i: (0, 0)),
            pl.BlockSpec((1, ncls), lambda i: (0, 0)),
        ],
        out_specs=pl.BlockSpec((rows, ncls), lambda i: (i, 0)),
        out_shape=jax.ShapeDtypeStruct((N, ncls), f32),
    )(pn, dcnt, b.reshape(1, HID), g.reshape(1, HID), be.reshape(1, HID),
      rm.reshape(1, HID), rv.reshape(1, HID), wc1, bc1.reshape(1, h1),
      wc2, bc2.reshape(1, ncls))


def kernel(x, hyperedge_index, W0, b0, g0, be0, rm0, rv0,
           W1, b1, g1, be1, rm1, rv1, W2, b2, g2, be2, rm2, rv2,
           Wc1, bc1, Wc2, bc2):
    pad_s = jnp.asarray(_PAD_SCAT)
    pad_g = jnp.asarray(_PAD_GATH)

    def prep(idx, pad):
        return jnp.concatenate([idx.reshape(NW, PER_W), pad], axis=1).reshape(
            NW, NCHUNK, CHUNK)

    s_src = prep(hyperedge_index[0], pad_s)
    g_src = prep(hyperedge_index[0], pad_g)
    s_he = prep(hyperedge_index[1], pad_s)
    g_he = prep(hyperedge_index[1], pad_g)
    zeros64 = jnp.zeros((MP, HID), f32)
    zeros16 = jnp.zeros((MP, 16), f32)
    ones16 = jnp.ones((CHUNK, 16), f32)

    dcnt, ecnt = _hist_call()(s_src, s_he, zeros16, ones16)
    stage1 = _stage1_call()

    params = [(b0, g0, be0, rm0, rv0),
              (b1, g1, be1, rm1, rv1),
              (b2, g2, be2, rm2, rv2)]
    next_w = [W1, W2]
    xw = _tc_matmul0(x, W0)
    for li, (b, g, be, rm, rv) in enumerate(params):
        pe = stage1(xw, g_src, s_he, zeros64)
        out_e = _tc_combine_e(pe, ecnt)
        pn = stage1(out_e, g_he, s_src, zeros64)
        if li < 2:
            xw = _tc_epilogue_matmul(pn, dcnt, b, g, be, rm, rv, next_w[li])
        else:
            return _tc_epilogue_head(pn, dcnt, b, g, be, rm, rv,
                                     Wc1, bc1, Wc2, bc2)
